# Initial kernel scaffold; baseline (speedup 1.0000x reference)
import jax
import jax.numpy as jnp
from jax.experimental import pallas as pl


def kernel(embeddings, w, past_sol, mask, W1, b1, W2, b2, Wc, bc, i):
    raise NotImplementedError("write your pallas kernel here")


# speedup vs baseline: 2.1244x; 2.1244x over previous
import functools

import jax
import jax.numpy as jnp
from jax import lax
from jax.experimental import pallas as pl
from jax.experimental.pallas import tpu as pltpu
from jax.experimental.pallas import tpu_sc as plsc

BN = 2048
UN = 100
VN = 100
DN = 128
IN_ = 50
STEPN = UN + 1 + IN_
U1 = UN + 1
IM1 = IN_ - 1

NC = 2
NS = 16
NW = NC * NS
BPW = BN // NW
CB = 2
NCHUNK = BPW // CB
CROWS = CB * IM1
GPS = 4
SROWS = GPS * CROWS
NSUP = NCHUNK // GPS


def _sc_gather(eflat, idx3):
    mesh = plsc.VectorSubcoreMesh(core_axis_name="c", subcore_axis_name="s")

    @functools.partial(
        pl.kernel,
        mesh=mesh,
        out_type=jax.ShapeDtypeStruct((BN * IM1, DN), jnp.float32),
        scratch_types=[
            pltpu.VMEM((NCHUNK, CROWS), jnp.int32),
            pltpu.VMEM((2, SROWS, DN), jnp.float32),
            pltpu.SemaphoreType.DMA,
            pltpu.SemaphoreType.DMA,
            pltpu.SemaphoreType.DMA,
            pltpu.SemaphoreType.DMA,
            pltpu.SemaphoreType.DMA,
        ],
    )
    def k(ef_hbm, idx_hbm, out_hbm, idx_v, stage, sem_i, gs0, gs1, os0, os1):
        gsem = (gs0, gs1)
        osem = (os0, os1)
        wid = lax.axis_index("s") * NC + lax.axis_index("c")
        rbase = wid * BPW * IM1
        pltpu.sync_copy(idx_hbm.at[wid], idx_v)

        def gfire(s, buf):
            cps = []
            for q in range(GPS):
                cp = pltpu.make_async_copy(
                    ef_hbm.at[idx_v.at[s * GPS + q]],
                    stage.at[buf].at[pl.ds(q * CROWS, CROWS)], gsem[buf])
                cp.start()
                cps.append(cp)
            return cps

        def ostart(s, buf):
            cp = pltpu.make_async_copy(
                stage.at[buf],
                out_hbm.at[pl.ds(rbase + s * SROWS, SROWS)], osem[buf])
            cp.start()
            return cp

        gs, os_ = {}, {}
        gs[0] = gfire(0, 0)
        for s in range(NSUP):
            nxt = s + 1
            if nxt < NSUP:
                if nxt >= 2:
                    os_[nxt - 2].wait()
                gs[nxt] = gfire(nxt, nxt % 2)
            for cp in gs[s]:
                cp.wait()
            os_[s] = ostart(s, s % 2)
        os_[NSUP - 2].wait()
        os_[NSUP - 1].wait()

    return k(eflat, idx3)


BB = 64
NG = BN // BB


def _bf(x):
    return x.astype(jnp.bfloat16).astype(jnp.float32)


def _tc_body(emb_ref, w_ref, mask_ref, sel_ref_in, w1emb_ref, w1icm_ref,
             w1w_ref, w1i_ref, b1_ref, if_ref, w2_ref, b2_ref, wct_ref,
             wcb_ref, bc_ref, p_ref, sel_ref, ll_ref, ent_ref):
    ib = pl.program_id(0)
    E = emb_ref[...]
    mean_all = jnp.mean(E, axis=1)
    incoming = E[:, STEPN - 1, :]
    Emid = E[:, U1:U1 + IM1, :]

    dgn = (((2,), (0,)), ((), ()))
    ctx_pre = (lax.dot_general(sel_ref_in[...], wct_ref[...], dgn,
                               preferred_element_type=jnp.float32)
               + lax.dot_general(Emid, wcb_ref[...], dgn,
                                 preferred_element_type=jnp.float32)
               + bc_ref[...][None])
    ctx = jnp.mean(ctx_pre, axis=1)

    cat = jnp.concatenate([incoming, ctx, mean_all], axis=1)
    base = jnp.dot(cat, w1icm_ref[...], preferred_element_type=jnp.float32)
    base = base + b1_ref[...] + _bf(if_ref[0, 0]) * _bf(w1i_ref[...])

    Eu = E[:, :U1, :]
    pre = lax.dot_general(Eu, w1emb_ref[...], dgn,
                          preferred_element_type=jnp.float32)
    pre = (pre + base[:, None, :]
           + _bf(w_ref[...])[:, :, None] * _bf(w1w_ref[...])[None])
    h = jnp.maximum(pre, 0.0)
    pi = lax.dot_general(h, w2_ref[...], dgn,
                         preferred_element_type=jnp.float32)
    pi = pi[:, :, 0] + b2_ref[0, 0]
    pi = jnp.where(mask_ref[...], -1e6, pi)

    m = jnp.max(pi, axis=1, keepdims=True)
    ex = jnp.exp(pi - m)
    se = jnp.sum(ex, axis=1, keepdims=True)
    lse = jnp.log(se)
    p = pi - m - lse
    p_ref[...] = p
    sel_ref[...] = jnp.argmax(pi, axis=1).astype(jnp.int32)[:, None]
    ll_ref[...] = -lse
    ent_term = jnp.sum(p * (ex / se))

    @pl.when(ib == 0)
    def _():
        ent_ref[...] = jnp.zeros((1, 1), jnp.float32)

    ent_ref[...] += (ent_term * (-1.0 / BN)).reshape(1, 1)


def _tc_head(emb, w, mask, selected, w1emb, w1icm, w1w, w1i, b1, if32, w2,
             b2, wct, wcb, bc):
    full = lambda shp: pl.BlockSpec(shp, lambda ib: (0,) * len(shp))
    grid_spec = pl.GridSpec(
        grid=(NG,),
        in_specs=[
            pl.BlockSpec((BB, STEPN, DN), lambda ib: (ib, 0, 0)),
            pl.BlockSpec((BB, U1), lambda ib: (ib, 0)),
            pl.BlockSpec((BB, U1), lambda ib: (ib, 0)),
            pl.BlockSpec((BB, IM1, DN), lambda ib: (ib, 0, 0)),
            full((DN, 200)),
            full((3 * DN, 200)),
            full((1, 200)),
            full((1, 200)),
            full((1, 200)),
            full((1, 1)),
            full((200, 1)),
            full((1, 1)),
            full((DN, DN)),
            full((DN, DN)),
            full((1, DN)),
        ],
        out_specs=[
            pl.BlockSpec((BB, U1), lambda ib: (ib, 0)),
            pl.BlockSpec((BB, 1), lambda ib: (ib, 0)),
            pl.BlockSpec((BB, 1), lambda ib: (ib, 0)),
            pl.BlockSpec((1, 1), lambda ib: (0, 0)),
        ],
    )
    return pl.pallas_call(
        _tc_body,
        grid_spec=grid_spec,
        out_shape=[
            jax.ShapeDtypeStruct((BN, U1), jnp.float32),
            jax.ShapeDtypeStruct((BN, 1), jnp.int32),
            jax.ShapeDtypeStruct((BN, 1), jnp.float32),
            jax.ShapeDtypeStruct((1, 1), jnp.float32),
        ],
    )(emb, w, mask, selected, w1emb, w1icm, w1w, w1i, b1, if32, w2, b2,
      wct, wcb, bc)


def kernel(embeddings, w, past_sol, mask, W1, b1, W2, b2, Wc, bc, i):
    eflat = embeddings.reshape(BN * STEPN, DN)
    offsets = jnp.arange(0, BN * IM1, IM1, dtype=past_sol.dtype)[:, None]
    idx3 = (past_sol + offsets).reshape(NW, NCHUNK, CROWS)

    selected = _sc_gather(eflat, idx3).reshape(BN, IM1, DN)

    w1w = W1[0:1, :]
    w1i = W1[1:2, :]
    w1emb = W1[2 + DN:2 + 2 * DN, :]
    w1icm = jnp.concatenate(
        [W1[2:2 + DN, :], W1[2 + 2 * DN:2 + 3 * DN, :],
         W1[2 + 3 * DN:2 + 4 * DN, :]], axis=0)
    if32 = (jnp.asarray(i).astype(jnp.float32) / jnp.float32(VN)).reshape(1, 1)
    b2s = b2.reshape(1, 1)
    b1r = b1.reshape(1, 200)
    bcr = bc.reshape(1, DN)
    wct = Wc[:DN, :]
    wcb = Wc[DN:, :]

    p, sel2, ll2, ent = _tc_head(embeddings, w, mask, selected, w1emb, w1icm,
                                 w1w, w1i, b1r, if32, W2, b2s, wct, wcb, bcr)
    return p, sel2.reshape(BN), ll2.reshape(BN), ent.reshape(())


# speedup vs baseline: 2.4457x; 1.1512x over previous
import functools

import jax
import jax.numpy as jnp
from jax import lax
from jax.experimental import pallas as pl
from jax.experimental.pallas import tpu as pltpu
from jax.experimental.pallas import tpu_sc as plsc

BN = 2048
UN = 100
VN = 100
DN = 128
IN_ = 50
STEPN = UN + 1 + IN_
U1 = UN + 1
IM1 = IN_ - 1

NC = 2
NS = 16
NW = NC * NS
BPW = BN // NW
CB = 2
NCHUNK = BPW // CB
CROWS = CB * IM1
GPS = 4
SROWS = GPS * CROWS
NSUP = NCHUNK // GPS


def _sc_gather(eflat, idx3):
    mesh = plsc.VectorSubcoreMesh(core_axis_name="c", subcore_axis_name="s")

    @functools.partial(
        pl.kernel,
        mesh=mesh,
        out_type=jax.ShapeDtypeStruct((BN * IM1, DN), jnp.float32),
        scratch_types=[
            pltpu.VMEM((NCHUNK, CROWS), jnp.int32),
            pltpu.VMEM((2, SROWS, DN), jnp.float32),
            pltpu.SemaphoreType.DMA,
            pltpu.SemaphoreType.DMA,
            pltpu.SemaphoreType.DMA,
            pltpu.SemaphoreType.DMA,
            pltpu.SemaphoreType.DMA,
        ],
    )
    def k(ef_hbm, idx_hbm, out_hbm, idx_v, stage, sem_i, gs0, gs1, os0, os1):
        gsem = (gs0, gs1)
        osem = (os0, os1)
        wid = lax.axis_index("s") * NC + lax.axis_index("c")
        rbase = wid * BPW * IM1
        pltpu.sync_copy(idx_hbm.at[wid], idx_v)

        def gfire(s, buf):
            cps = []
            for q in range(GPS):
                cp = pltpu.make_async_copy(
                    ef_hbm.at[idx_v.at[s * GPS + q]],
                    stage.at[buf].at[pl.ds(q * CROWS, CROWS)], gsem[buf])
                cp.start()
                cps.append(cp)
            return cps

        def ostart(s, buf):
            cp = pltpu.make_async_copy(
                stage.at[buf],
                out_hbm.at[pl.ds(rbase + s * SROWS, SROWS)], osem[buf])
            cp.start()
            return cp

        gs, os_ = {}, {}
        gs[0] = gfire(0, 0)
        for s in range(NSUP):
            nxt = s + 1
            if nxt < NSUP:
                if nxt >= 2:
                    os_[nxt - 2].wait()
                gs[nxt] = gfire(nxt, nxt % 2)
            for cp in gs[s]:
                cp.wait()
            os_[s] = ostart(s, s % 2)
        os_[NSUP - 2].wait()
        os_[NSUP - 1].wait()

    return k(eflat, idx3)


BB = 64
NG = BN // BB


def _bf(x):
    return x.astype(jnp.bfloat16).astype(jnp.float32)


def _tc_body(emb_ref, w_ref, mask_ref, sel_ref_in, w1emb_ref, w1icm_ref,
             w1w_ref, w1i_ref, b1_ref, if_ref, w2_ref, b2_ref, wct_ref,
             wcb_ref, bc_ref, p_ref, sel_ref, ll_ref, ent_ref):
    ib = pl.program_id(0)
    E = emb_ref[...]
    mean_all = jnp.mean(E, axis=1)
    incoming = E[:, STEPN - 1, :]
    Emid = E[:, U1:U1 + IM1, :]

    bf16 = jnp.bfloat16
    dgn = (((2,), (0,)), ((), ()))
    ctx_pre = (lax.dot_general(sel_ref_in[...].astype(bf16), wct_ref[...],
                               dgn, preferred_element_type=jnp.float32)
               + lax.dot_general(Emid.astype(bf16), wcb_ref[...], dgn,
                                 preferred_element_type=jnp.float32)
               + bc_ref[...][None])
    ctx = jnp.mean(ctx_pre, axis=1)

    cat = jnp.concatenate([incoming, ctx, mean_all], axis=1)
    base = jnp.dot(cat.astype(bf16), w1icm_ref[...],
                   preferred_element_type=jnp.float32)
    base = base + b1_ref[...] + _bf(if_ref[0, 0]) * _bf(w1i_ref[...])

    Eu = E[:, :U1, :]
    pre = lax.dot_general(Eu.astype(bf16), w1emb_ref[...], dgn,
                          preferred_element_type=jnp.float32)
    pre = (pre + base[:, None, :]
           + _bf(w_ref[...])[:, :, None] * _bf(w1w_ref[...])[None])
    h = jnp.maximum(pre, 0.0)
    pi = jnp.sum(_bf(h) * _bf(w2_ref[...])[None], axis=2)
    pi = pi + b2_ref[0, 0]
    pi = jnp.where(mask_ref[...], -1e6, pi)

    m = jnp.max(pi, axis=1, keepdims=True)
    ex = jnp.exp(pi - m)
    se = jnp.sum(ex, axis=1, keepdims=True)
    lse = jnp.log(se)
    p = pi - m - lse
    p_ref[...] = p
    sel_ref[...] = jnp.argmax(pi, axis=1).astype(jnp.int32)[:, None]
    ll_ref[...] = -lse
    ent_term = jnp.sum(p * (ex / se))

    @pl.when(ib == 0)
    def _():
        ent_ref[...] = jnp.zeros((1, 1), jnp.float32)

    ent_ref[...] += (ent_term * (-1.0 / BN)).reshape(1, 1)


def _tc_head(emb, w, mask, selected, w1emb, w1icm, w1w, w1i, b1, if32, w2,
             b2, wct, wcb, bc):
    full = lambda shp: pl.BlockSpec(shp, lambda ib: (0,) * len(shp))
    grid_spec = pl.GridSpec(
        grid=(NG,),
        in_specs=[
            pl.BlockSpec((BB, STEPN, DN), lambda ib: (ib, 0, 0)),
            pl.BlockSpec((BB, U1), lambda ib: (ib, 0)),
            pl.BlockSpec((BB, U1), lambda ib: (ib, 0)),
            pl.BlockSpec((BB, IM1, DN), lambda ib: (ib, 0, 0)),
            full((DN, 200)),
            full((3 * DN, 200)),
            full((1, 200)),
            full((1, 200)),
            full((1, 200)),
            full((1, 1)),
            full((1, 200)),
            full((1, 1)),
            full((DN, DN)),
            full((DN, DN)),
            full((1, DN)),
        ],
        out_specs=[
            pl.BlockSpec((BB, U1), lambda ib: (ib, 0)),
            pl.BlockSpec((BB, 1), lambda ib: (ib, 0)),
            pl.BlockSpec((BB, 1), lambda ib: (ib, 0)),
            pl.BlockSpec((1, 1), lambda ib: (0, 0)),
        ],
    )
    return pl.pallas_call(
        _tc_body,
        grid_spec=grid_spec,
        out_shape=[
            jax.ShapeDtypeStruct((BN, U1), jnp.float32),
            jax.ShapeDtypeStruct((BN, 1), jnp.int32),
            jax.ShapeDtypeStruct((BN, 1), jnp.float32),
            jax.ShapeDtypeStruct((1, 1), jnp.float32),
        ],
    )(emb, w, mask, selected, w1emb, w1icm, w1w, w1i, b1, if32, w2, b2,
      wct, wcb, bc)


def kernel(embeddings, w, past_sol, mask, W1, b1, W2, b2, Wc, bc, i):
    eflat = embeddings.reshape(BN * STEPN, DN)
    offsets = jnp.arange(0, BN * IM1, IM1, dtype=past_sol.dtype)[:, None]
    idx3 = (past_sol + offsets).reshape(NW, NCHUNK, CROWS)

    selected = _sc_gather(eflat, idx3).reshape(BN, IM1, DN)

    w1w = W1[0:1, :]
    w1i = W1[1:2, :]
    w1emb = W1[2 + DN:2 + 2 * DN, :].astype(jnp.bfloat16)
    w1icm = jnp.concatenate(
        [W1[2:2 + DN, :], W1[2 + 2 * DN:2 + 3 * DN, :],
         W1[2 + 3 * DN:2 + 4 * DN, :]], axis=0).astype(jnp.bfloat16)
    if32 = (jnp.asarray(i).astype(jnp.float32) / jnp.float32(VN)).reshape(1, 1)
    b2s = b2.reshape(1, 1)
    b1r = b1.reshape(1, 200)
    bcr = bc.reshape(1, DN)
    wct = Wc[:DN, :].astype(jnp.bfloat16)
    wcb = Wc[DN:, :].astype(jnp.bfloat16)

    w2r = W2.reshape(1, 200)
    p, sel2, ll2, ent = _tc_head(embeddings, w, mask, selected, w1emb, w1icm,
                                 w1w, w1i, b1r, if32, w2r, b2s, wct, wcb, bcr)
    return p, sel2.reshape(BN), ll2.reshape(BN), ent.reshape(())


# speedup vs baseline: 4.5739x; 1.8702x over previous
import functools

import jax
import jax.numpy as jnp
from jax import lax
from jax.experimental import pallas as pl
from jax.experimental.pallas import tpu as pltpu
from jax.experimental.pallas import tpu_sc as plsc

BN = 2048
UN = 100
VN = 100
DN = 128
IN_ = 50
STEPN = UN + 1 + IN_
U1 = UN + 1
IM1 = IN_ - 1
SP = 152
JP = 56
UP = 104
NQ = (2047 * IM1 + UN) // STEPN + 1

NC = 2
NS = 16
NW = NC * NS
BPW = BN // NW
CB = 2
NCHUNK = BPW // CB
CROWS = CB * JP
GPS = 4
SROWS = GPS * CROWS
NSUP = NCHUNK // GPS


def _sc_gather(epad, idx3):
    mesh = plsc.VectorSubcoreMesh(core_axis_name="c", subcore_axis_name="s")

    @functools.partial(
        pl.kernel,
        mesh=mesh,
        out_type=jax.ShapeDtypeStruct((BN * JP, DN), jnp.float32),
        scratch_types=[
            pltpu.VMEM((NCHUNK, CROWS), jnp.int32),
            pltpu.VMEM((2, SROWS, DN), jnp.float32),
            pltpu.SemaphoreType.DMA,
            pltpu.SemaphoreType.DMA,
            pltpu.SemaphoreType.DMA,
            pltpu.SemaphoreType.DMA,
            pltpu.SemaphoreType.DMA,
        ],
    )
    def k(ef_hbm, idx_hbm, out_hbm, idx_v, stage, sem_i, gs0, gs1, os0, os1):
        gsem = (gs0, gs1)
        osem = (os0, os1)
        wid = lax.axis_index("s") * NC + lax.axis_index("c")
        rbase = wid * BPW * JP
        pltpu.sync_copy(idx_hbm.at[wid], idx_v)

        def gfire(s, buf):
            cps = []
            for q in range(GPS):
                cp = pltpu.make_async_copy(
                    ef_hbm.at[idx_v.at[s * GPS + q]],
                    stage.at[buf].at[pl.ds(q * CROWS, CROWS)], gsem[buf])
                cp.start()
                cps.append(cp)
            return cps

        def ostart(s, buf):
            cp = pltpu.make_async_copy(
                stage.at[buf],
                out_hbm.at[pl.ds(rbase + s * SROWS, SROWS)], osem[buf])
            cp.start()
            return cp

        gs, os_ = {}, {}
        gs[0] = gfire(0, 0)
        for s in range(NSUP):
            nxt = s + 1
            if nxt < NSUP:
                if nxt >= 2:
                    os_[nxt - 2].wait()
                gs[nxt] = gfire(nxt, nxt % 2)
            for cp in gs[s]:
                cp.wait()
            os_[s] = ostart(s, s % 2)
        os_[NSUP - 2].wait()
        os_[NSUP - 1].wait()

    return k(epad, idx3)


BB = 64
NG = BN // BB


def _bf(x):
    return x.astype(jnp.bfloat16).astype(jnp.float32)


def _tc_body(emb_ref, w_ref, mask_ref, selp_ref, w1emb_ref, w1icm_ref,
             w1w_ref, w1i_ref, b1_ref, if_ref, w2_ref, b2_ref, wct_ref,
             wcb_ref, bc_ref, p_ref, sel_ref, ll_ref, ent_ref):
    ib = pl.program_id(0)
    bf16 = jnp.bfloat16
    E3 = emb_ref[...]
    ii = lax.broadcasted_iota(jnp.int32, (1, SP, 1), 1)
    mean_all = jnp.sum(jnp.where(ii < STEPN, E3, 0.0),
                       axis=1) * (1.0 / STEPN)
    incoming = E3[:, STEPN - 1, :]

    E2b = E3.reshape(BB * SP, DN).astype(bf16)
    A3 = jnp.dot(E2b, w1emb_ref[...],
                 preferred_element_type=jnp.float32).reshape(BB, SP, 200)
    B3 = jnp.dot(E2b, wcb_ref[...],
                 preferred_element_type=jnp.float32).reshape(BB, SP, DN)
    mid_sum = jnp.sum(
        jnp.where((ii >= U1) & (ii < STEPN - 1), B3, 0.0), axis=1)

    S2b = selp_ref[...].astype(bf16)
    S3 = jnp.dot(S2b, wct_ref[...],
                 preferred_element_type=jnp.float32).reshape(BB, JP, DN)
    jj = lax.broadcasted_iota(jnp.int32, (1, JP, 1), 1)
    sel_sum = jnp.sum(jnp.where(jj < IM1, S3, 0.0), axis=1)

    ctx = (sel_sum + mid_sum + float(IM1) * bc_ref[...]) * (1.0 / IM1)

    cat = jnp.concatenate([incoming, ctx, mean_all], axis=1)
    base = jnp.dot(cat.astype(bf16), w1icm_ref[...],
                   preferred_element_type=jnp.float32)
    base = base + b1_ref[...] + _bf(if_ref[0, 0]) * _bf(w1i_ref[...])

    pre = (A3[:, :UP, :] + base[:, None, :]
           + _bf(w_ref[...])[:, :, None] * _bf(w1w_ref[...])[None])
    h = jnp.maximum(pre, 0.0)
    pi = jnp.sum(_bf(h) * _bf(w2_ref[...])[None], axis=2)
    pi = pi[:, :U1] + b2_ref[0, 0]
    pi = jnp.where(mask_ref[...], -1e6, pi)

    m = jnp.max(pi, axis=1, keepdims=True)
    ex = jnp.exp(pi - m)
    se = jnp.sum(ex, axis=1, keepdims=True)
    lse = jnp.log(se)
    p = pi - m - lse
    p_ref[...] = p
    sel_ref[...] = jnp.argmax(pi, axis=1).astype(jnp.int32)[:, None]
    ll_ref[...] = -lse
    ent_term = jnp.sum(p * (ex / se))

    @pl.when(ib == 0)
    def _():
        ent_ref[...] = jnp.zeros((1, 1), jnp.float32)

    ent_ref[...] += (ent_term * (-1.0 / BN)).reshape(1, 1)


def _tc_head(emb, w104, mask, selpad, w1emb, w1icm, w1w, w1i, b1, if32, w2,
             b2, wct, wcb, bc):
    full = lambda shp: pl.BlockSpec(shp, lambda ib: (0,) * len(shp))
    grid_spec = pl.GridSpec(
        grid=(NG,),
        in_specs=[
            pl.BlockSpec((BB, SP, DN), lambda ib: (ib, 0, 0)),
            pl.BlockSpec((BB, UP), lambda ib: (ib, 0)),
            pl.BlockSpec((BB, U1), lambda ib: (ib, 0)),
            pl.BlockSpec((BB * JP, DN), lambda ib: (ib, 0)),
            full((DN, 200)),
            full((3 * DN, 200)),
            full((1, 200)),
            full((1, 200)),
            full((1, 200)),
            full((1, 1)),
            full((1, 200)),
            full((1, 1)),
            full((DN, DN)),
            full((DN, DN)),
            full((1, DN)),
        ],
        out_specs=[
            pl.BlockSpec((BB, U1), lambda ib: (ib, 0)),
            pl.BlockSpec((BB, 1), lambda ib: (ib, 0)),
            pl.BlockSpec((BB, 1), lambda ib: (ib, 0)),
            pl.BlockSpec((1, 1), lambda ib: (0, 0)),
        ],
    )
    return pl.pallas_call(
        _tc_body,
        grid_spec=grid_spec,
        out_shape=[
            jax.ShapeDtypeStruct((BN, U1), jnp.float32),
            jax.ShapeDtypeStruct((BN, 1), jnp.int32),
            jax.ShapeDtypeStruct((BN, 1), jnp.float32),
            jax.ShapeDtypeStruct((1, 1), jnp.float32),
        ],
    )(emb, w104, mask, selpad, w1emb, w1icm, w1w, w1i, b1, if32, w2, b2,
      wct, wcb, bc)


def kernel(embeddings, w, past_sol, mask, W1, b1, W2, b2, Wc, bc, i):
    epad = jnp.pad(embeddings[:NQ], ((0, 0), (0, SP - STEPN), (0, 0))
                   ).reshape(NQ * SP, DN)
    offsets = jnp.arange(0, BN * IM1, IM1, dtype=past_sol.dtype)[:, None]
    fidx = past_sol + offsets
    q, s = jnp.divmod(fidx, STEPN)
    idx152 = q * SP + s
    idxp = jnp.concatenate(
        [idx152, jnp.broadcast_to(idx152[:, :1], (BN, JP - IM1))], axis=1)
    idx3 = idxp.reshape(NW, NCHUNK, CROWS)

    selpad = _sc_gather(epad, idx3)

    w104 = jnp.pad(w, ((0, 0), (0, UP - U1)))
    w1w = W1[0:1, :]
    w1i = W1[1:2, :]
    w1emb = W1[2 + DN:2 + 2 * DN, :].astype(jnp.bfloat16)
    w1icm = jnp.concatenate(
        [W1[2:2 + DN, :], W1[2 + 2 * DN:2 + 3 * DN, :],
         W1[2 + 3 * DN:2 + 4 * DN, :]], axis=0).astype(jnp.bfloat16)
    if32 = (jnp.asarray(i).astype(jnp.float32) / jnp.float32(VN)).reshape(1, 1)
    b2s = b2.reshape(1, 1)
    b1r = b1.reshape(1, 200)
    bcr = bc.reshape(1, DN)
    wct = Wc[:DN, :].astype(jnp.bfloat16)
    wcb = Wc[DN:, :].astype(jnp.bfloat16)
    w2r = W2.reshape(1, 200)

    p, sel2, ll2, ent = _tc_head(embeddings, w104, mask, selpad, w1emb,
                                 w1icm, w1w, w1i, b1r, if32, w2r, b2s, wct,
                                 wcb, bcr)
    return p, sel2.reshape(BN), ll2.reshape(BN), ent.reshape(())


# speedup vs baseline: 9.1208x; 1.9941x over previous
import functools

import jax
import jax.numpy as jnp
from jax import lax
from jax.experimental import pallas as pl
from jax.experimental.pallas import tpu as pltpu
from jax.experimental.pallas import tpu_sc as plsc

BN = 2048
UN = 100
VN = 100
DN = 128
IN_ = 50
STEPN = UN + 1 + IN_
U1 = UN + 1
IM1 = IN_ - 1

NC = 2
NS = 16
NW = NC * NS
BPW = BN // NW
CB = 2
NCHUNK = BPW // CB
CROWS = CB * IM1


def _sc_gather(etab, idx3, oidx3):
    mesh = plsc.VectorSubcoreMesh(core_axis_name="c", subcore_axis_name="s")

    @functools.partial(
        pl.kernel,
        mesh=mesh,
        out_type=jax.ShapeDtypeStruct((IM1 * BN, DN), jnp.float32),
        scratch_types=[
            pltpu.VMEM((NCHUNK, CROWS), jnp.int32),
            pltpu.VMEM((NCHUNK, CROWS), jnp.int32),
            pltpu.VMEM((2, CROWS, DN), jnp.float32),
            pltpu.SemaphoreType.DMA,
            pltpu.SemaphoreType.DMA,
            pltpu.SemaphoreType.DMA,
            pltpu.SemaphoreType.DMA,
        ],
    )
    def k(ef_hbm, idx_hbm, oidx_hbm, out_hbm, idx_v, oidx_v, stage,
          gs0, gs1, ss0, ss1):
        gsem = (gs0, gs1)
        ssem = (ss0, ss1)
        wid = lax.axis_index("s") * NC + lax.axis_index("c")
        pltpu.sync_copy(idx_hbm.at[wid], idx_v)
        pltpu.sync_copy(oidx_hbm.at[wid], oidx_v)

        def gstart(c, buf):
            cp = pltpu.make_async_copy(
                ef_hbm.at[idx_v.at[c]], stage.at[buf], gsem[buf])
            cp.start()
            return cp

        def sstart(c, buf):
            cp = pltpu.make_async_copy(
                stage.at[buf], out_hbm.at[oidx_v.at[c]], ssem[buf])
            cp.start()
            return cp

        gs, ss = {}, {}
        gs[0] = gstart(0, 0)
        for c in range(NCHUNK):
            gs[c].wait()
            ss[c] = sstart(c, c % 2)
            nxt = c + 1
            if nxt < NCHUNK:
                if nxt >= 2:
                    ss[nxt - 2].wait()
                gs[nxt] = gstart(nxt, nxt % 2)
        ss[NCHUNK - 2].wait()
        ss[NCHUNK - 1].wait()

    return k(etab, idx3, oidx3)


BB = 128
NG = BN // BB
MU = U1 * BB
MM = IM1 * BB


def _bf(x):
    return x.astype(jnp.bfloat16).astype(jnp.float32)


def _tc_body(emb_ref, wt_ref, mask_ref, selp_ref, w1emb_ref, w1icm_ref,
             w1w_ref, w1i_ref, b1_ref, if_ref, w2_ref, b2_ref, wct_ref,
             wcb_ref, bc_ref, p_ref, sel_ref, ll_ref, ent_ref):
    ib = pl.program_id(0)
    bf16 = jnp.bfloat16
    E3 = emb_ref[...]
    mean_all = jnp.sum(E3, axis=0) * (1.0 / STEPN)
    incoming = E3[STEPN - 1]

    E2b = E3.reshape(STEPN * BB, DN).astype(bf16)
    A3 = jnp.dot(E2b[:MU], w1emb_ref[...],
                 preferred_element_type=jnp.float32).reshape(U1, BB, 200)
    mid_sum = jnp.sum(
        jnp.dot(E2b[MU:MU + MM], wcb_ref[...],
                preferred_element_type=jnp.float32).reshape(IM1, BB, DN),
        axis=0)
    S2b = selp_ref[...].reshape(MM, DN).astype(bf16)
    sel_sum = jnp.sum(
        jnp.dot(S2b, wct_ref[...],
                preferred_element_type=jnp.float32).reshape(IM1, BB, DN),
        axis=0)
    ctx = (sel_sum + mid_sum + float(IM1) * bc_ref[...]) * (1.0 / IM1)

    cat = jnp.concatenate([incoming, ctx, mean_all], axis=1)
    base = jnp.dot(cat.astype(bf16), w1icm_ref[...],
                   preferred_element_type=jnp.float32)
    base = base + b1_ref[...] + _bf(if_ref[0, 0]) * _bf(w1i_ref[...])

    wcol3 = wt_ref[...].reshape(U1, BB, 1)
    pre = (A3 + base[None]
           + _bf(wcol3) * _bf(w1w_ref[...])[None])
    h = jnp.maximum(pre, 0.0)
    pi = jnp.sum(_bf(h) * _bf(w2_ref[...])[None], axis=2)
    pi = pi + b2_ref[0, 0]
    pi = jnp.where(mask_ref[...], -1e6, pi)

    m = jnp.max(pi, axis=0, keepdims=True)
    ex = jnp.exp(pi - m)
    se = jnp.sum(ex, axis=0, keepdims=True)
    lse = jnp.log(se)
    p = pi - m - lse
    p_ref[...] = p
    sel_ref[...] = jnp.argmax(pi, axis=0).astype(jnp.int32)[None]
    ll_ref[...] = -lse
    ent_term = jnp.sum(p * (ex / se))

    @pl.when(ib == 0)
    def _():
        ent_ref[...] = jnp.zeros((1, 1), jnp.float32)

    ent_ref[...] += (ent_term * (-1.0 / BN)).reshape(1, 1)


def _tc_head(embT, wT, maskT, selT3, w1emb, w1icm, w1w, w1i, b1, if32, w2,
             b2, wct, wcb, bc):
    full = lambda shp: pl.BlockSpec(shp, lambda ib: (0,) * len(shp))
    grid_spec = pl.GridSpec(
        grid=(NG,),
        in_specs=[
            pl.BlockSpec((STEPN, BB, DN), lambda ib: (0, ib, 0)),
            pl.BlockSpec((U1, BB), lambda ib: (0, ib)),
            pl.BlockSpec((U1, BB), lambda ib: (0, ib)),
            pl.BlockSpec((IM1, BB, DN), lambda ib: (0, ib, 0)),
            full((DN, 200)),
            full((3 * DN, 200)),
            full((1, 200)),
            full((1, 200)),
            full((1, 200)),
            full((1, 1)),
            full((1, 200)),
            full((1, 1)),
            full((DN, DN)),
            full((DN, DN)),
            full((1, DN)),
        ],
        out_specs=[
            pl.BlockSpec((U1, BB), lambda ib: (0, ib)),
            pl.BlockSpec((1, BB), lambda ib: (0, ib)),
            pl.BlockSpec((1, BB), lambda ib: (0, ib)),
            pl.BlockSpec((1, 1), lambda ib: (0, 0)),
        ],
    )
    return pl.pallas_call(
        _tc_body,
        grid_spec=grid_spec,
        out_shape=[
            jax.ShapeDtypeStruct((U1, BN), jnp.float32),
            jax.ShapeDtypeStruct((1, BN), jnp.int32),
            jax.ShapeDtypeStruct((1, BN), jnp.float32),
            jax.ShapeDtypeStruct((1, 1), jnp.float32),
        ],
    )(embT, wT, maskT, selT3, w1emb, w1icm, w1w, w1i, b1, if32, w2, b2,
      wct, wcb, bc)


def kernel(embeddings, w, past_sol, mask, W1, b1, W2, b2, Wc, bc, i):
    embT = jnp.transpose(embeddings, (1, 0, 2))
    etab = embT.reshape(STEPN * BN, DN)
    offsets = jnp.arange(0, BN * IM1, IM1, dtype=past_sol.dtype)[:, None]
    fidx = past_sol + offsets
    q, s = jnp.divmod(fidx, STEPN)
    gidx = s * BN + q
    idx3 = gidx.reshape(NW, NCHUNK, CROWS)
    oidx = (jnp.arange(IM1, dtype=jnp.int32)[None, :] * BN
            + jnp.arange(BN, dtype=jnp.int32)[:, None])
    oidx3 = oidx.reshape(NW, NCHUNK, CROWS)

    selT3 = _sc_gather(etab, idx3, oidx3).reshape(IM1, BN, DN)

    wT = jnp.transpose(w)
    maskT = jnp.transpose(mask)
    w1w = W1[0:1, :]
    w1i = W1[1:2, :]
    w1emb = W1[2 + DN:2 + 2 * DN, :].astype(jnp.bfloat16)
    w1icm = jnp.concatenate(
        [W1[2:2 + DN, :], W1[2 + 2 * DN:2 + 3 * DN, :],
         W1[2 + 3 * DN:2 + 4 * DN, :]], axis=0).astype(jnp.bfloat16)
    if32 = (jnp.asarray(i).astype(jnp.float32) / jnp.float32(VN)).reshape(1, 1)
    b2s = b2.reshape(1, 1)
    b1r = b1.reshape(1, 200)
    bcr = bc.reshape(1, DN)
    wct = Wc[:DN, :].astype(jnp.bfloat16)
    wcb = Wc[DN:, :].astype(jnp.bfloat16)
    w2r = W2.reshape(1, 200)

    pT, selL, llL, ent = _tc_head(embT, wT, maskT, selT3, w1emb, w1icm,
                                  w1w, w1i, b1r, if32, w2r, b2s, wct, wcb,
                                  bcr)
    return (jnp.transpose(pT), selL.reshape(BN), llL.reshape(BN),
            ent.reshape(()))


# speedup vs baseline: 9.7103x; 1.0646x over previous
import functools

import jax
import jax.numpy as jnp
from jax import lax
from jax.experimental import pallas as pl
from jax.experimental.pallas import tpu as pltpu
from jax.experimental.pallas import tpu_sc as plsc

BN = 2048
UN = 100
VN = 100
DN = 128
IN_ = 50
STEPN = UN + 1 + IN_
U1 = UN + 1
IM1 = IN_ - 1

NC = 2
NS = 16
NW = NC * NS
BPW = BN // NW
CB = 2
NCHUNK = BPW // CB
CROWS = CB * IM1


def _sc_gather(etab, idx3, oidx3, nout):
    nchunk = idx3.shape[1]
    mesh = plsc.VectorSubcoreMesh(core_axis_name="c", subcore_axis_name="s")

    @functools.partial(
        pl.kernel,
        mesh=mesh,
        out_type=jax.ShapeDtypeStruct((nout, DN), jnp.float32),
        scratch_types=[
            pltpu.VMEM((nchunk, CROWS), jnp.int32),
            pltpu.VMEM((nchunk, CROWS), jnp.int32),
            pltpu.VMEM((2, CROWS, DN), jnp.float32),
            pltpu.SemaphoreType.DMA,
            pltpu.SemaphoreType.DMA,
            pltpu.SemaphoreType.DMA,
            pltpu.SemaphoreType.DMA,
        ],
    )
    def k(ef_hbm, idx_hbm, oidx_hbm, out_hbm, idx_v, oidx_v, stage,
          gs0, gs1, ss0, ss1):
        gsem = (gs0, gs1)
        ssem = (ss0, ss1)
        wid = lax.axis_index("s") * NC + lax.axis_index("c")
        pltpu.sync_copy(idx_hbm.at[wid], idx_v)
        pltpu.sync_copy(oidx_hbm.at[wid], oidx_v)

        def gstart(c, buf):
            cp = pltpu.make_async_copy(
                ef_hbm.at[idx_v.at[c]], stage.at[buf], gsem[buf])
            cp.start()
            return cp

        def sstart(c, buf):
            cp = pltpu.make_async_copy(
                stage.at[buf], out_hbm.at[oidx_v.at[c]], ssem[buf])
            cp.start()
            return cp

        gs, ss = {}, {}
        gs[0] = gstart(0, 0)
        for c in range(nchunk):
            gs[c].wait()
            ss[c] = sstart(c, c % 2)
            nxt = c + 1
            if nxt < nchunk:
                if nxt >= 2:
                    ss[nxt - 2].wait()
                gs[nxt] = gstart(nxt, nxt % 2)
        ss[nchunk - 2].wait()
        ss[nchunk - 1].wait()

    return k(etab, idx3, oidx3)


BB = 128
NG = BN // BB
MU = U1 * BB
MM = IM1 * BB


def _bf(x):
    return x.astype(jnp.bfloat16).astype(jnp.float32)


def _tc_body(emb_ref, wt_ref, mask_ref, selp_ref, w1emb_ref, w1icm_ref,
             w1w_ref, w1i_ref, b1_ref, if_ref, w2_ref, b2_ref, wct_ref,
             wcb_ref, bc_ref, p_ref, sel_ref, ll_ref, ent_ref):
    ib = pl.program_id(0)
    bf16 = jnp.bfloat16
    E3 = emb_ref[...]
    mean_all = jnp.sum(E3, axis=0) * (1.0 / STEPN)
    incoming = E3[STEPN - 1]

    E2b = E3.reshape(STEPN * BB, DN).astype(bf16)
    A3 = jnp.dot(E2b[:MU], w1emb_ref[...],
                 preferred_element_type=jnp.float32).reshape(U1, BB, 200)
    mid_sum = jnp.sum(
        jnp.dot(E2b[MU:MU + MM], wcb_ref[...],
                preferred_element_type=jnp.float32).reshape(IM1, BB, DN),
        axis=0)
    S2b = selp_ref[...].reshape(MM, DN).astype(bf16)
    sel_sum = jnp.sum(
        jnp.dot(S2b, wct_ref[...],
                preferred_element_type=jnp.float32).reshape(IM1, BB, DN),
        axis=0)
    ctx = (sel_sum + mid_sum + float(IM1) * bc_ref[...]) * (1.0 / IM1)

    cat = jnp.concatenate([incoming, ctx, mean_all], axis=1)
    base = jnp.dot(cat.astype(bf16), w1icm_ref[...],
                   preferred_element_type=jnp.float32)
    base = base + b1_ref[...] + _bf(if_ref[0, 0]) * _bf(w1i_ref[...])

    wcol3 = wt_ref[...].reshape(U1, BB, 1)
    pre = (A3 + base[None]
           + _bf(wcol3) * _bf(w1w_ref[...])[None])
    h = jnp.maximum(pre, 0.0)
    pi = jnp.sum(_bf(h) * _bf(w2_ref[...])[None], axis=2)
    pi = pi + b2_ref[0, 0]
    pi = jnp.where(mask_ref[...], -1e6, pi)

    m = jnp.max(pi, axis=0, keepdims=True)
    ex = jnp.exp(pi - m)
    se = jnp.sum(ex, axis=0, keepdims=True)
    lse = jnp.log(se)
    p = pi - m - lse
    p_ref[...] = p
    sel_ref[...] = jnp.argmax(pi, axis=0).astype(jnp.int32)[None]
    ll_ref[...] = -lse
    ent_term = jnp.sum(p * (ex / se))

    @pl.when(ib == 0)
    def _():
        ent_ref[...] = jnp.zeros((1, 1), jnp.float32)

    ent_ref[...] += ent_term.reshape(1, 1)


def _tc_head(embT, wT, maskT, selT3, w1emb, w1icm, w1w, w1i, b1, if32, w2,
             b2, wct, wcb, bc, boff, bnh):
    ng = bnh // BB
    ob = boff // BB
    full = lambda shp: pl.BlockSpec(shp, lambda ib: (0,) * len(shp))
    grid_spec = pl.GridSpec(
        grid=(ng,),
        in_specs=[
            pl.BlockSpec((STEPN, BB, DN), lambda ib: (0, ob + ib, 0)),
            pl.BlockSpec((U1, BB), lambda ib: (0, ob + ib)),
            pl.BlockSpec((U1, BB), lambda ib: (0, ob + ib)),
            pl.BlockSpec((IM1, BB, DN), lambda ib: (0, ib, 0)),
            full((DN, 200)),
            full((3 * DN, 200)),
            full((1, 200)),
            full((1, 200)),
            full((1, 200)),
            full((1, 1)),
            full((1, 200)),
            full((1, 1)),
            full((DN, DN)),
            full((DN, DN)),
            full((1, DN)),
        ],
        out_specs=[
            pl.BlockSpec((U1, BB), lambda ib: (0, ib)),
            pl.BlockSpec((1, BB), lambda ib: (0, ib)),
            pl.BlockSpec((1, BB), lambda ib: (0, ib)),
            pl.BlockSpec((1, 1), lambda ib: (0, 0)),
        ],
    )
    return pl.pallas_call(
        _tc_body,
        grid_spec=grid_spec,
        out_shape=[
            jax.ShapeDtypeStruct((U1, bnh), jnp.float32),
            jax.ShapeDtypeStruct((1, bnh), jnp.int32),
            jax.ShapeDtypeStruct((1, bnh), jnp.float32),
            jax.ShapeDtypeStruct((1, 1), jnp.float32),
        ],
    )(embT, wT, maskT, selT3, w1emb, w1icm, w1w, w1i, b1, if32, w2, b2,
      wct, wcb, bc)


def kernel(embeddings, w, past_sol, mask, W1, b1, W2, b2, Wc, bc, i):
    embT = jnp.transpose(embeddings, (1, 0, 2))
    etab = embT.reshape(STEPN * BN, DN)
    offsets = jnp.arange(0, BN * IM1, IM1, dtype=past_sol.dtype)[:, None]
    fidx = past_sol + offsets
    q, s = jnp.divmod(fidx, STEPN)
    gidx = s * BN + q
    idx3 = gidx.reshape(NW, NCHUNK, CROWS)
    BH = BN // 2
    oidx = (jnp.arange(IM1, dtype=jnp.int32)[None, :] * BH
            + (jnp.arange(BN, dtype=jnp.int32) % BH)[:, None])
    oidx3 = oidx.reshape(NW, NCHUNK, CROWS)

    idxh = idx3.reshape(2, NW, NCHUNK // 2, CROWS)
    oidxh = oidx3.reshape(2, NW, NCHUNK // 2, CROWS)
    selA = _sc_gather(etab, idxh[0], oidxh[0], IM1 * BH).reshape(IM1, BH, DN)
    selB = _sc_gather(etab, idxh[1], oidxh[1], IM1 * BH).reshape(IM1, BH, DN)

    wT = jnp.transpose(w)
    maskT = jnp.transpose(mask)
    w1w = W1[0:1, :]
    w1i = W1[1:2, :]
    w1emb = W1[2 + DN:2 + 2 * DN, :].astype(jnp.bfloat16)
    w1icm = jnp.concatenate(
        [W1[2:2 + DN, :], W1[2 + 2 * DN:2 + 3 * DN, :],
         W1[2 + 3 * DN:2 + 4 * DN, :]], axis=0).astype(jnp.bfloat16)
    if32 = (jnp.asarray(i).astype(jnp.float32) / jnp.float32(VN)).reshape(1, 1)
    b2s = b2.reshape(1, 1)
    b1r = b1.reshape(1, 200)
    bcr = bc.reshape(1, DN)
    wct = Wc[:DN, :].astype(jnp.bfloat16)
    wcb = Wc[DN:, :].astype(jnp.bfloat16)
    w2r = W2.reshape(1, 200)

    pA, sA, lA, eA = _tc_head(embT, wT, maskT, selA, w1emb, w1icm, w1w,
                              w1i, b1r, if32, w2r, b2s, wct, wcb, bcr,
                              0, BH)
    pB, sB, lB, eB = _tc_head(embT, wT, maskT, selB, w1emb, w1icm, w1w,
                              w1i, b1r, if32, w2r, b2s, wct, wcb, bcr,
                              BH, BH)
    pT = jnp.concatenate([pA, pB], axis=1)
    sel = jnp.concatenate([sA, sB], axis=1).reshape(BN)
    ll = jnp.concatenate([lA, lB], axis=1).reshape(BN)
    ent = (eA + eB).reshape(()) * (-1.0 / BN)
    return jnp.transpose(pT), sel, ll, ent
